# baseline (speedup 1.0000x reference)
import functools

import jax
import jax.numpy as jnp
from jax import lax
from jax.experimental import pallas as pl
from jax.experimental.pallas import tpu as pltpu
from jax.experimental.pallas import tpu_sc as plsc

NUM_EXPERTS = 64
TOP_K = 2
BLOCK_S = 4096

B = 4
S = 8192
N_WORKERS = 32
CHUNK = (B * S) // N_WORKERS
CHUNKS_PER_BATCH = S // CHUNK


def _gate_body(x_ref, w_ref, logits_ref):
    x = x_ref[0]
    w = w_ref[...]
    logits_ref[0] = jax.lax.dot_general(
        w, x, (((1,), (1,)), ((), ())), preferred_element_type=jnp.float32
    )


def _gate_matmul(hidden_states, gate_w):
    b, s, h = hidden_states.shape
    grid = (b, s // BLOCK_S)
    return pl.pallas_call(
        _gate_body,
        grid=grid,
        in_specs=[
            pl.BlockSpec((1, BLOCK_S, h), lambda i, j: (i, j, 0)),
            pl.BlockSpec((NUM_EXPERTS, h), lambda i, j: (0, 0)),
        ],
        out_specs=pl.BlockSpec((1, NUM_EXPERTS, BLOCK_S), lambda i, j: (i, 0, j)),
        out_shape=jax.ShapeDtypeStruct((b, NUM_EXPERTS, s), jnp.float32),
    )(hidden_states, gate_w)


def _topk_sc_body(logits_hbm, w_out, i_out, buf, wbuf, ibuf):
    cid = lax.axis_index("c")
    sid = lax.axis_index("s")
    wid = sid * 2 + cid
    batch = wid // CHUNKS_PER_BATCH
    tok0 = (wid % CHUNKS_PER_BATCH) * CHUNK

    pltpu.sync_copy(logits_hbm.at[batch, :, pl.ds(tok0, CHUNK)], buf)

    def group(g, _):
        t = pl.multiple_of(g * 16, 16)
        m1 = jnp.full((16,), -jnp.inf, jnp.float32)
        m2 = jnp.full((16,), -jnp.inf, jnp.float32)
        i1 = jnp.zeros((16,), jnp.int32)
        i2 = jnp.zeros((16,), jnp.int32)
        for e in range(NUM_EXPERTS):
            v = buf[e, pl.ds(t, 16)]
            e_vec = jnp.full((16,), e, jnp.int32)
            is1 = v > m1
            is2 = v > m2
            m2n = jnp.where(is2, v, m2)
            i2n = jnp.where(is2, e_vec, i2)
            m2 = jnp.where(is1, m1, m2n)
            i2 = jnp.where(is1, i1, i2n)
            m1 = jnp.where(is1, v, m1)
            i1 = jnp.where(is1, e_vec, i1)
        ex = jnp.exp(m2 - m1)
        w0 = 1.0 / (1.0 + ex)
        wbuf[0, pl.ds(t, 16)] = w0
        wbuf[1, pl.ds(t, 16)] = 1.0 - w0
        ibuf[0, pl.ds(t, 16)] = i1
        ibuf[1, pl.ds(t, 16)] = i2
        return _

    lax.fori_loop(0, CHUNK // 16, group, None)

    pltpu.sync_copy(wbuf, w_out.at[batch, :, pl.ds(tok0, CHUNK)])
    pltpu.sync_copy(ibuf, i_out.at[batch, :, pl.ds(tok0, CHUNK)])


def _topk_sc(logits_t):
    mesh = plsc.VectorSubcoreMesh(core_axis_name="c", subcore_axis_name="s")
    fn = functools.partial(
        pl.kernel,
        mesh=mesh,
        out_type=[
            jax.ShapeDtypeStruct((B, TOP_K, S), jnp.float32),
            jax.ShapeDtypeStruct((B, TOP_K, S), jnp.int32),
        ],
        scratch_types=[
            pltpu.VMEM((NUM_EXPERTS, CHUNK), jnp.float32),
            pltpu.VMEM((TOP_K, CHUNK), jnp.float32),
            pltpu.VMEM((TOP_K, CHUNK), jnp.int32),
        ],
    )(_topk_sc_body)
    return fn(logits_t)


@jax.jit
def kernel(hidden_states, gate_w):
    b, s, h = hidden_states.shape
    logits_t = _gate_matmul(hidden_states, gate_w)
    weights_t, idx_t = _topk_sc(logits_t)
    return (
        jnp.transpose(weights_t, (0, 2, 1)),
        jnp.transpose(idx_t, (0, 2, 1)),
        jnp.transpose(logits_t, (0, 2, 1)),
    )


# speedup vs baseline: 1.7946x; 1.7946x over previous
import jax
import jax.numpy as jnp
from jax.experimental import pallas as pl

NUM_EXPERTS = 64
TOP_K = 2
BLOCK_S = 4096


def _router_body(x_ref, w_ref, logits_ref, weights_ref, idx_ref):
    x = x_ref[0]
    w = w_ref[...]
    lt = jax.lax.dot_general(
        w, x, (((1,), (1,)), ((), ())), preferred_element_type=jnp.float32
    )
    logits_ref[0] = lt

    eid = jax.lax.broadcasted_iota(jnp.int32, lt.shape, 0)
    m1 = jnp.max(lt, axis=0, keepdims=True)
    i1 = jnp.min(jnp.where(lt == m1, eid, NUM_EXPERTS), axis=0, keepdims=True)
    masked = jnp.where(eid == i1, -jnp.inf, lt)
    m2 = jnp.max(masked, axis=0, keepdims=True)
    i2 = jnp.min(
        jnp.where(masked == m2, eid, NUM_EXPERTS), axis=0, keepdims=True
    )
    e = jnp.exp(m2 - m1)
    w0 = 1.0 / (1.0 + e)
    weights_ref[0] = jnp.concatenate([w0, 1.0 - w0], axis=0)
    idx_ref[0] = jnp.concatenate([i1, i2], axis=0)


@jax.jit
def kernel(hidden_states, gate_w):
    b, s, h = hidden_states.shape
    grid = (b, s // BLOCK_S)
    logits_t, weights_t, idx_t = pl.pallas_call(
        _router_body,
        grid=grid,
        in_specs=[
            pl.BlockSpec((1, BLOCK_S, h), lambda i, j: (i, j, 0)),
            pl.BlockSpec((NUM_EXPERTS, h), lambda i, j: (0, 0)),
        ],
        out_specs=[
            pl.BlockSpec((1, NUM_EXPERTS, BLOCK_S), lambda i, j: (i, 0, j)),
            pl.BlockSpec((1, TOP_K, BLOCK_S), lambda i, j: (i, 0, j)),
            pl.BlockSpec((1, TOP_K, BLOCK_S), lambda i, j: (i, 0, j)),
        ],
        out_shape=[
            jax.ShapeDtypeStruct((b, NUM_EXPERTS, s), jnp.float32),
            jax.ShapeDtypeStruct((b, TOP_K, s), jnp.float32),
            jax.ShapeDtypeStruct((b, TOP_K, s), jnp.int32),
        ],
    )(hidden_states, gate_w)
    return (
        jnp.transpose(weights_t, (0, 2, 1)),
        jnp.transpose(idx_t, (0, 2, 1)),
        jnp.transpose(logits_t, (0, 2, 1)),
    )
